# baseline (speedup 1.0000x reference)
import functools

import jax
import jax.numpy as jnp
from jax import lax
from jax.experimental import pallas as pl
from jax.experimental.pallas import tpu as pltpu
from jax.experimental.pallas import tpu_sc as plsc

_B = 4096
_L = 200
_D = 32
_V = 1000000
_NL = 10
_NW = 32
_BPW = _B // _NW
_C0 = 128
_C1 = _L - _C0
_LP = 208

_CB = 512
_NB = (_V + _CB - 1) // _CB

_mesh = plsc.VectorSubcoreMesh(core_axis_name="c", subcore_axis_name="s")


def _format_body(in_ref, out_ref):
    blk = in_ref[...]
    t = blk.T
    rr = lax.broadcasted_iota(jnp.int32, (_CB // 4, _CB), 0)
    cc = lax.broadcasted_iota(jnp.int32, (_CB // 4, _CB), 1)
    for a in range(4):
        sel = jnp.where(cc == 4 * rr + a, 1.0, 0.0)
        out_ref[:, pl.ds(32 * a, 32)] = jnp.dot(
            sel, t, preferred_element_type=jnp.float32)


def _format_table(tableT):
    return pl.pallas_call(
        _format_body,
        out_shape=jax.ShapeDtypeStruct((_V // 4, 128), jnp.float32),
        grid=(_NB,),
        in_specs=[pl.BlockSpec((_D, _CB), lambda i: (0, i))],
        out_specs=pl.BlockSpec((_CB // 4, 128), lambda i: (i, 0)),
    )(tableT)


@functools.partial(
    pl.kernel,
    out_type=jax.ShapeDtypeStruct((_B, _D), jnp.float32),
    mesh=_mesh,
    scratch_types=[
        pltpu.VMEM((_BPW * _L,), jnp.int32),
        pltpu.VMEM((_BPW * _LP,), jnp.int32),
        pltpu.VMEM((2, _L, 128), jnp.float32),
        pltpu.VMEM((_BPW, _D), jnp.float32),
        pltpu.SemaphoreType.DMA,
        pltpu.SemaphoreType.DMA,
    ],
    compiler_params=pltpu.CompilerParams(
        use_tc_tiling_on_sc=False, needs_layout_passes=False),
)
def _embed_sum(xg_hbm, xq_hbm, table_hbm, doc_hbm,
               idx_v, xq_v, rows_v, doc_v, sem0, sem1):
    wid = lax.axis_index("s") * 2 + lax.axis_index("c")
    base = wid * _BPW
    pltpu.sync_copy(xg_hbm.at[pl.ds(base * _L, _BPW * _L)], idx_v)
    pltpu.sync_copy(xq_hbm.at[pl.ds(wid * (_BPW * _LP), _BPW * _LP)], xq_v)

    sems = (sem0, sem1)

    def descs(r, p, sem):
        d0 = pltpu.make_async_copy(
            table_hbm.at[idx_v.at[pl.ds(r * _L, _C0)]],
            rows_v.at[p, pl.ds(0, _C0)], sem)
        d1 = pltpu.make_async_copy(
            table_hbm.at[idx_v.at[pl.ds(r * _L + _C0, _C1)]],
            rows_v.at[p, pl.ds(_C0, _C1)], sem)
        return d0, d1

    def issue(r, p):
        d0, d1 = descs(r, p, sems[p])
        d0.start()
        d1.start()

    issue(0, 0)
    issue(1, 1)

    zeros = jnp.zeros((16,), jnp.float32)
    iota = lax.iota(jnp.int32, 16)

    def splat_lane(vec, lane):
        return lax.gather(
            vec, jnp.full((16, 1), lane, jnp.int32),
            lax.GatherDimensionNumbers(
                offset_dims=(), collapsed_slice_dims=(0,),
                start_index_map=(0,)),
            (1,), mode=lax.GatherScatterMode.PROMISE_IN_BOUNDS)

    def outer(g, carry):
        for p in range(2):
            r = g * 2 + p
            d0, d1 = descs(r, p, sems[p])
            d0.wait()
            d1.wait()

            splat_p = jnp.full((16,), p, jnp.int32)

            def group(jbase, qchunk, nu, acc):
                a0, a1, b0, b1 = acc
                for u in range(nu):
                    jvec = jnp.full((16,), u, jnp.int32) + jbase
                    q = splat_lane(qchunk, u)
                    lo = plsc.load_gather(rows_v, [splat_p, jvec, q + iota])
                    hi = plsc.load_gather(rows_v, [splat_p, jvec, q + 16 + iota])
                    if u % 2 == 0:
                        a0 = a0 + lo
                        a1 = a1 + hi
                    else:
                        b0 = b0 + lo
                        b1 = b1 + hi
                return (a0, a1, b0, b1)

            def rbody(jj, acc):
                jbase = jj * 16
                qchunk = xq_v[pl.ds(r * _LP + jbase, 16)]
                return group(jbase, qchunk, 16, acc)

            acc = lax.fori_loop(
                0, _L // 16, rbody, (zeros, zeros, zeros, zeros))
            a0, a1, b0, b1 = group(
                192, xq_v[pl.ds(r * _LP + 192, 16)], _L - 192, acc)

            @pl.when(r + 2 < _BPW)
            def _():
                issue(r + 2, p)

            doc_v[r, pl.ds(0, 16)] = a0 + b0
            doc_v[r, pl.ds(16, 16)] = a1 + b1
        return carry

    lax.fori_loop(0, _BPW // 2, outer, 0)
    pltpu.sync_copy(doc_v, doc_hbm.at[pl.ds(base, _BPW)])


def _head_body(doc_ref, w_ref, b_ref, out_ref):
    out_ref[...] = (
        jnp.dot(doc_ref[...], w_ref[...], preferred_element_type=jnp.float32)
        + b_ref[...]
    )


def _head(doc, W, b):
    nblk = 4
    return pl.pallas_call(
        _head_body,
        out_shape=jax.ShapeDtypeStruct((_B, _NL), jnp.float32),
        grid=(nblk,),
        in_specs=[
            pl.BlockSpec((_B // nblk, _D), lambda i: (i, 0)),
            pl.BlockSpec((_D, _NL), lambda i: (0, 0)),
            pl.BlockSpec((1, _NL), lambda i: (0, 0)),
        ],
        out_specs=pl.BlockSpec((_B // nblk, _NL), lambda i: (i, 0)),
    )(doc, W, b.reshape(1, _NL))


def kernel(x, m, table, W, b):
    del m
    x = x.astype(jnp.int32)
    xg = (x >> 2).reshape(-1)
    xq = jnp.pad((x & 3) << 5, ((0, 0), (0, _LP - _L))).reshape(-1)
    t2 = _format_table(table.T)
    doc = _embed_sum(xg, xq, t2)
    return _head(doc, W, b)
